# Initial kernel scaffold; baseline (speedup 1.0000x reference)
import functools

import jax
import jax.numpy as jnp
from jax import lax
from jax.experimental import pallas as pl
from jax.experimental.pallas import tpu as pltpu
from jax.experimental.pallas import tpu_sc as plsc

N = 10000
E = 320000
D = 128
C = 40

NC = 2
NS = 16
L = 16
NW = NC * NS
EPW = E // NW
K = 80
CHUNKS = EPW // K
RPW = N // NS
ZR = 125

_mesh = plsc.VectorSubcoreMesh(core_axis_name="c", subcore_axis_name="s")


@functools.partial(
    pl.kernel,
    out_type=jax.ShapeDtypeStruct((NC, N, D), jnp.float32),
    mesh=_mesh,
    scratch_types=[
        pltpu.VMEM((CHUNKS, K), jnp.int32),
        pltpu.VMEM((CHUNKS, K), jnp.int32),
        pltpu.VMEM((CHUNKS, K), jnp.float32),
        pltpu.VMEM((K, D), jnp.float32),
        pltpu.VMEM((ZR, D), jnp.float32),
        pltpu.VMEM_SHARED((N, D), jnp.float32),
        pltpu.SemaphoreType.DMA,
    ],
)
def _sc_aggregate(x_hbm, src_hbm, dst_hbm, val_hbm, out_hbm,
                  srcv, dstv, valv, rows, zbuf, acc, gsem):
    c = lax.axis_index("c")
    s = lax.axis_index("s")
    wid = c * NS + s

    zv = jnp.zeros((L,), jnp.float32)

    @pl.loop(0, ZR)
    def _(i):
        for j in range(D // L):
            zbuf[i, pl.ds(j * L, L)] = zv

    @pl.loop(0, RPW // ZR)
    def _(t):
        pltpu.sync_copy(zbuf, acc.at[pl.ds(s * RPW + t * ZR, ZR)])

    pltpu.sync_copy(src_hbm.at[wid], srcv)
    pltpu.sync_copy(dst_hbm.at[wid], dstv)
    pltpu.sync_copy(val_hbm.at[wid], valv)

    plsc.subcore_barrier()

    @pl.loop(0, CHUNKS)
    def _(g):
        pltpu.async_copy(x_hbm.at[srcv.at[g]], rows, gsem).wait()
        gfull = jnp.full((L,), g, jnp.int32)

        @pl.loop(0, K)
        def _(e):
            v = plsc.load_gather(valv, [gfull, jnp.full((L,), e, jnp.int32)])
            for j in range(D // L):
                sl = (e, pl.ds(j * L, L))
                rows[sl] = rows[sl] * v

        pltpu.sync_copy(rows, acc.at[dstv.at[g]], add=True)

    plsc.subcore_barrier()

    pltpu.sync_copy(acc.at[pl.ds(s * RPW, RPW)],
                    out_hbm.at[c, pl.ds(s * RPW, RPW)])


BN = 1000


def _mm_relu_body(p_ref, w_ref, o_ref):
    h = p_ref[0] + p_ref[1]
    y = lax.dot_general(h, w_ref[...], (((1,), (0,)), ((), ())),
                        preferred_element_type=jnp.float32,
                        precision=lax.Precision.HIGHEST)
    o_ref[...] = jnp.maximum(y, 0.0)


def _tc_mm_relu(p, w):
    return pl.pallas_call(
        _mm_relu_body,
        grid=(N // BN,),
        in_specs=[
            pl.BlockSpec((NC, BN, D), lambda i: (0, i, 0)),
            pl.BlockSpec((D, D), lambda i: (0, 0)),
        ],
        out_specs=pl.BlockSpec((BN, D), lambda i: (i, 0)),
        out_shape=jax.ShapeDtypeStruct((N, D), jnp.float32),
    )(p, w)


def _final_body(p_ref, w_ref, c_ref, o_ref):
    h = p_ref[0] + p_ref[1]
    y = lax.dot_general(h, w_ref[...], (((1,), (0,)), ((), ())),
                        preferred_element_type=jnp.float32,
                        precision=lax.Precision.HIGHEST)
    h2 = jnp.maximum(y, 0.0)
    o_ref[...] = lax.dot_general(h2, c_ref[...], (((1,), (0,)), ((), ())),
                                 preferred_element_type=jnp.float32,
                                 precision=lax.Precision.HIGHEST)


def _tc_final(p, w, cls):
    return pl.pallas_call(
        _final_body,
        grid=(N // BN,),
        in_specs=[
            pl.BlockSpec((NC, BN, D), lambda i: (0, i, 0)),
            pl.BlockSpec((D, D), lambda i: (0, 0)),
            pl.BlockSpec((D, D), lambda i: (0, 0)),
        ],
        out_specs=pl.BlockSpec((BN, D), lambda i: (i, 0)),
        out_shape=jax.ShapeDtypeStruct((N, D), jnp.float32),
    )(p, w, cls)


def kernel(x, edge_index, edge_val, w0, w1, classifier):
    src = edge_index[0].reshape(NW, CHUNKS, K)
    dst = edge_index[1].reshape(NW, CHUNKS, K)
    val = edge_val.reshape(NW, CHUNKS, K)
    cls_pad = jnp.zeros((D, D), jnp.float32).at[:, :C].set(classifier)

    p1 = _sc_aggregate(x, src, dst, val)
    h1 = _tc_mm_relu(p1, w0)
    p2 = _sc_aggregate(h1, src, dst, val)
    out = _tc_final(p2, w1, cls_pad)
    return out[:, :C]


# speedup vs baseline: 3.7085x; 3.7085x over previous
import dataclasses
import functools

import jax
import jax.numpy as jnp
from jax import lax
from jax.experimental import pallas as pl
from jax.experimental.pallas import tpu as pltpu
from jax.experimental.pallas import tpu_sc as plsc

N = 10000
E = 320000
D = 128
C = 40

NC = 2
NS = 16
L = 16
D2 = D // NC
EPS = E // NS
K = 80
CHUNKS = EPS // K
RPW = 624
TAIL = N - NS * RPW
ZR = 156

_mesh = plsc.VectorSubcoreMesh(core_axis_name="c", subcore_axis_name="s")

_sc_params = pltpu.CompilerParams(
    needs_layout_passes=False, use_tc_tiling_on_sc=False)


@functools.partial(
    pl.kernel,
    out_type=jax.ShapeDtypeStruct((NC * N, D2), jnp.float32),
    mesh=_mesh,
    scratch_types=[
        pltpu.VMEM((CHUNKS, K), jnp.int32),
        pltpu.VMEM((CHUNKS, K), jnp.int32),
        pltpu.VMEM((CHUNKS, K), jnp.float32),
        pltpu.VMEM((K, D2), jnp.float32),
        pltpu.VMEM((ZR, D2), jnp.float32),
        pltpu.VMEM_SHARED((N, D2), jnp.float32),
        pltpu.SemaphoreType.DMA,
    ],
    compiler_params=_sc_params,
)
def _sc_aggregate(x_hbm, src_hbm, dst_hbm, val_hbm, out_hbm,
                  srcv, dstv, valv, rows, zbuf, acc, gsem):
    c = lax.axis_index("c")
    s = lax.axis_index("s")

    zv = jnp.zeros((L,), jnp.float32)

    @pl.loop(0, ZR)
    def _(i):
        for j in range(D2 // L):
            zbuf[i, pl.ds(j * L, L)] = zv

    @pl.loop(0, RPW // ZR)
    def _(t):
        pltpu.sync_copy(zbuf, acc.at[pl.ds(s * RPW + t * ZR, ZR)])

    @pl.when(s == 0)
    def _():
        pltpu.sync_copy(zbuf.at[pl.ds(0, TAIL)], acc.at[pl.ds(NS * RPW, TAIL)])

    pltpu.sync_copy(src_hbm.at[s], srcv)
    pltpu.sync_copy(dst_hbm.at[s], dstv)
    pltpu.sync_copy(val_hbm.at[s], valv)

    coff = jnp.full((L,), c * N, jnp.int32)

    @pl.loop(0, CHUNKS)
    def _(g):
        for t in range(K // L):
            sl = (g, pl.ds(t * L, L))
            srcv[sl] = srcv[sl] + coff

    plsc.subcore_barrier()

    @pl.loop(0, CHUNKS)
    def _(g):
        pltpu.async_copy(x_hbm.at[srcv.at[g]], rows, gsem).wait()
        gfull = jnp.full((L,), g, jnp.int32)

        @pl.loop(0, K)
        def _(e):
            v = plsc.load_gather(valv, [gfull, jnp.full((L,), e, jnp.int32)])
            for j in range(D2 // L):
                sl = (e, pl.ds(j * L, L))
                rows[sl] = rows[sl] * v

        pltpu.sync_copy(rows, acc.at[dstv.at[g]], add=True)

    plsc.subcore_barrier()

    pltpu.sync_copy(acc.at[pl.ds(s * RPW, RPW)],
                    out_hbm.at[pl.ds(c * N + s * RPW, RPW)])

    @pl.when(s == 0)
    def _():
        pltpu.sync_copy(acc.at[pl.ds(NS * RPW, TAIL)],
                        out_hbm.at[pl.ds(c * N + NS * RPW, TAIL)])


BN = 1000
NB = N // BN


def _mm_relu_body(p0_ref, p1_ref, w_ref, o_ref):
    wv = w_ref[...]
    y = lax.dot_general(p0_ref[...], wv[:D2], (((1,), (0,)), ((), ())),
                        preferred_element_type=jnp.float32,
                        precision=lax.Precision.HIGHEST)
    y += lax.dot_general(p1_ref[...], wv[D2:], (((1,), (0,)), ((), ())),
                         preferred_element_type=jnp.float32,
                         precision=lax.Precision.HIGHEST)
    h = jnp.maximum(y, 0.0)
    o_ref[0] = h[:, :D2]
    o_ref[1] = h[:, D2:]


def _tc_mm_relu(p, w):
    return pl.pallas_call(
        _mm_relu_body,
        grid=(NB,),
        in_specs=[
            pl.BlockSpec((BN, D2), lambda i: (i, 0)),
            pl.BlockSpec((BN, D2), lambda i: (i + NB, 0)),
            pl.BlockSpec((D, D), lambda i: (0, 0)),
        ],
        out_specs=pl.BlockSpec((NC, BN, D2), lambda i: (0, i, 0)),
        out_shape=jax.ShapeDtypeStruct((NC, N, D2), jnp.float32),
    )(p, p, w)


def _final_body(p0_ref, p1_ref, w_ref, c_ref, o_ref):
    wv = w_ref[...]
    y = lax.dot_general(p0_ref[...], wv[:D2], (((1,), (0,)), ((), ())),
                        preferred_element_type=jnp.float32,
                        precision=lax.Precision.HIGHEST)
    y += lax.dot_general(p1_ref[...], wv[D2:], (((1,), (0,)), ((), ())),
                         preferred_element_type=jnp.float32,
                         precision=lax.Precision.HIGHEST)
    h = jnp.maximum(y, 0.0)
    o_ref[...] = lax.dot_general(h, c_ref[...], (((1,), (0,)), ((), ())),
                                 preferred_element_type=jnp.float32,
                                 precision=lax.Precision.HIGHEST)


def _tc_final(p, w, cls):
    return pl.pallas_call(
        _final_body,
        grid=(NB,),
        in_specs=[
            pl.BlockSpec((BN, D2), lambda i: (i, 0)),
            pl.BlockSpec((BN, D2), lambda i: (i + NB, 0)),
            pl.BlockSpec((D, D), lambda i: (0, 0)),
            pl.BlockSpec((D, D), lambda i: (0, 0)),
        ],
        out_specs=pl.BlockSpec((BN, D), lambda i: (i, 0)),
        out_shape=jax.ShapeDtypeStruct((N, D), jnp.float32),
    )(p, p, w, cls)


def kernel(x, edge_index, edge_val, w0, w1, classifier):
    src = edge_index[0].reshape(NS, CHUNKS, K)
    dst = edge_index[1].reshape(NS, CHUNKS, K)
    val = edge_val.reshape(NS, CHUNKS, K)
    cls_pad = jnp.zeros((D, D), jnp.float32).at[:, :C].set(classifier)
    xcat = jnp.concatenate([x[:, :D2], x[:, D2:]], axis=0)

    p1 = _sc_aggregate(xcat, src, dst, val)
    h1 = _tc_mm_relu(p1, w0)
    p2 = _sc_aggregate(h1.reshape(NC * N, D2), src, dst, val)
    out = _tc_final(p2, w1, cls_pad)
    return out[:, :C]


# speedup vs baseline: 3.9172x; 1.0563x over previous
import dataclasses
import functools

import jax
import jax.numpy as jnp
from jax import lax
from jax.experimental import pallas as pl
from jax.experimental.pallas import tpu as pltpu
from jax.experimental.pallas import tpu_sc as plsc

N = 10000
E = 320000
D = 128
C = 40

NC = 2
NS = 16
L = 16
D2 = D // NC
EPS = E // NS
K = 80
CHUNKS = EPS // K
RPW = 624
TAIL = N - NS * RPW
ZR = 156

_mesh = plsc.VectorSubcoreMesh(core_axis_name="c", subcore_axis_name="s")

_sc_params = pltpu.CompilerParams(
    needs_layout_passes=False, use_tc_tiling_on_sc=False)


@functools.partial(
    pl.kernel,
    out_type=jax.ShapeDtypeStruct((NC * N, D2), jnp.float32),
    mesh=_mesh,
    scratch_types=[
        pltpu.VMEM((CHUNKS, K), jnp.int32),
        pltpu.VMEM((CHUNKS, K), jnp.int32),
        pltpu.VMEM((CHUNKS, K), jnp.float32),
        pltpu.VMEM((K, D2), jnp.float32),
        pltpu.VMEM((K, D2), jnp.float32),
        pltpu.VMEM((ZR, D2), jnp.float32),
        pltpu.VMEM_SHARED((N, D2), jnp.float32),
        pltpu.SemaphoreType.DMA,
        pltpu.SemaphoreType.DMA,
    ],
    compiler_params=_sc_params,
)
def _sc_aggregate(x_hbm, src_hbm, dst_hbm, val_hbm, out_hbm,
                  srcv, dstv, valv, rows_a, rows_b, zbuf, acc, sem_a, sem_b):
    c = lax.axis_index("c")
    s = lax.axis_index("s")

    zv = jnp.zeros((L,), jnp.float32)

    @pl.loop(0, ZR)
    def _(i):
        for j in range(D2 // L):
            zbuf[i, pl.ds(j * L, L)] = zv

    @pl.loop(0, RPW // ZR)
    def _(t):
        pltpu.sync_copy(zbuf, acc.at[pl.ds(s * RPW + t * ZR, ZR)])

    @pl.when(s == 0)
    def _():
        pltpu.sync_copy(zbuf.at[pl.ds(0, TAIL)], acc.at[pl.ds(NS * RPW, TAIL)])

    pltpu.sync_copy(src_hbm.at[s], srcv)
    pltpu.sync_copy(dst_hbm.at[s], dstv)
    pltpu.sync_copy(val_hbm.at[s], valv)

    coff = jnp.full((L,), c * N, jnp.int32)

    @pl.loop(0, CHUNKS)
    def _(g):
        for t in range(K // L):
            sl = (g, pl.ds(t * L, L))
            srcv[sl] = srcv[sl] + coff

    plsc.subcore_barrier()

    bcast_dnums = lax.GatherDimensionNumbers(
        offset_dims=(), collapsed_slice_dims=(0,), start_index_map=(0,))

    def _scale(rows, g):
        @pl.loop(0, K // L)
        def _(q):
            val16 = valv[g, pl.ds(q * L, L)]
            for l in range(L):
                v = lax.gather(val16, jnp.full((L, 1), l, jnp.int32),
                               bcast_dnums, (1,),
                               mode=lax.GatherScatterMode.PROMISE_IN_BOUNDS)
                e = q * L + l
                for j in range(D2 // L):
                    sl = (e, pl.ds(j * L, L))
                    rows[sl] = rows[sl] * v

    pltpu.async_copy(x_hbm.at[srcv.at[0]], rows_a, sem_a)

    @pl.loop(0, CHUNKS // 2)
    def _(h):
        a = h * 2
        b = a + 1
        pltpu.make_async_copy(x_hbm.at[srcv.at[a]], rows_a, sem_a).wait()
        pltpu.async_copy(x_hbm.at[srcv.at[b]], rows_b, sem_b)
        _scale(rows_a, a)
        pltpu.sync_copy(rows_a, acc.at[dstv.at[a]], add=True)
        pltpu.make_async_copy(x_hbm.at[srcv.at[b]], rows_b, sem_b).wait()

        @pl.when(h < CHUNKS // 2 - 1)
        def _():
            pltpu.async_copy(x_hbm.at[srcv.at[a + 2]], rows_a, sem_a)

        _scale(rows_b, b)
        pltpu.sync_copy(rows_b, acc.at[dstv.at[b]], add=True)

    plsc.subcore_barrier()

    pltpu.sync_copy(acc.at[pl.ds(s * RPW, RPW)],
                    out_hbm.at[pl.ds(c * N + s * RPW, RPW)])

    @pl.when(s == 0)
    def _():
        pltpu.sync_copy(acc.at[pl.ds(NS * RPW, TAIL)],
                        out_hbm.at[pl.ds(c * N + NS * RPW, TAIL)])


BN = 1000
NB = N // BN


def _mm_relu_body(p0_ref, p1_ref, w_ref, o_ref):
    wv = w_ref[...]
    y = lax.dot_general(p0_ref[...], wv[:D2], (((1,), (0,)), ((), ())),
                        preferred_element_type=jnp.float32,
                        precision=lax.Precision.HIGHEST)
    y += lax.dot_general(p1_ref[...], wv[D2:], (((1,), (0,)), ((), ())),
                         preferred_element_type=jnp.float32,
                         precision=lax.Precision.HIGHEST)
    h = jnp.maximum(y, 0.0)
    o_ref[0] = h[:, :D2]
    o_ref[1] = h[:, D2:]


def _tc_mm_relu(p, w):
    return pl.pallas_call(
        _mm_relu_body,
        grid=(NB,),
        in_specs=[
            pl.BlockSpec((BN, D2), lambda i: (i, 0)),
            pl.BlockSpec((BN, D2), lambda i: (i + NB, 0)),
            pl.BlockSpec((D, D), lambda i: (0, 0)),
        ],
        out_specs=pl.BlockSpec((NC, BN, D2), lambda i: (0, i, 0)),
        out_shape=jax.ShapeDtypeStruct((NC, N, D2), jnp.float32),
    )(p, p, w)


def _final_body(p0_ref, p1_ref, w_ref, c_ref, o_ref):
    wv = w_ref[...]
    y = lax.dot_general(p0_ref[...], wv[:D2], (((1,), (0,)), ((), ())),
                        preferred_element_type=jnp.float32,
                        precision=lax.Precision.HIGHEST)
    y += lax.dot_general(p1_ref[...], wv[D2:], (((1,), (0,)), ((), ())),
                         preferred_element_type=jnp.float32,
                         precision=lax.Precision.HIGHEST)
    h = jnp.maximum(y, 0.0)
    o_ref[...] = lax.dot_general(h, c_ref[...], (((1,), (0,)), ((), ())),
                                 preferred_element_type=jnp.float32,
                                 precision=lax.Precision.HIGHEST)


def _tc_final(p, w, cls):
    return pl.pallas_call(
        _final_body,
        grid=(NB,),
        in_specs=[
            pl.BlockSpec((BN, D2), lambda i: (i, 0)),
            pl.BlockSpec((BN, D2), lambda i: (i + NB, 0)),
            pl.BlockSpec((D, D), lambda i: (0, 0)),
            pl.BlockSpec((D, D), lambda i: (0, 0)),
        ],
        out_specs=pl.BlockSpec((BN, D), lambda i: (i, 0)),
        out_shape=jax.ShapeDtypeStruct((N, D), jnp.float32),
    )(p, p, w, cls)


def kernel(x, edge_index, edge_val, w0, w1, classifier):
    src = edge_index[0].reshape(NS, CHUNKS, K)
    dst = edge_index[1].reshape(NS, CHUNKS, K)
    val = edge_val.reshape(NS, CHUNKS, K)
    cls_pad = jnp.zeros((D, D), jnp.float32).at[:, :C].set(classifier)
    xcat = jnp.concatenate([x[:, :D2], x[:, D2:]], axis=0)

    p1 = _sc_aggregate(xcat, src, dst, val)
    h1 = _tc_mm_relu(p1, w0)
    p2 = _sc_aggregate(h1.reshape(NC * N, D2), src, dst, val)
    out = _tc_final(p2, w1, cls_pad)
    return out[:, :C]
